# baseline (speedup 1.0000x reference)
import functools

import jax
import jax.numpy as jnp
from jax import lax
from jax.experimental import pallas as pl
from jax.experimental.pallas import tpu as pltpu
from jax.experimental.pallas import tpu_sc as plsc

N_ASPECTS = 20
EMBED_DIM = 64
BATCH = 16384
A_MIN, A_MAX = 1.0, 5.0
R_MIN, R_MAX = 1.0, 5.0


def _sc_gather(table, idx):
    info = plsc.get_sparse_core_info()
    NC, NS = info.num_cores, info.num_subcores
    NW = NC * NS
    B = idx.shape[0]
    D = table.shape[1]
    b_per_w = B // NW
    mesh = plsc.VectorSubcoreMesh(core_axis_name="c", subcore_axis_name="s")

    @functools.partial(
        pl.kernel,
        mesh=mesh,
        compiler_params=pltpu.CompilerParams(use_tc_tiling_on_sc=False),
        out_type=jax.ShapeDtypeStruct((B, D), jnp.float32),
        scratch_types=[
            pltpu.VMEM((b_per_w,), jnp.int32),
            pltpu.VMEM((b_per_w, D), jnp.float32),
            pltpu.SemaphoreType.DMA,
        ],
    )
    def k(table_hbm, idx_hbm, out_hbm, idx_v, rows_v, sem):
        wid = lax.axis_index("s") * NC + lax.axis_index("c")
        base = wid * b_per_w
        pltpu.sync_copy(idx_hbm.at[pl.ds(base, b_per_w)], idx_v)
        pltpu.async_copy(table_hbm.at[idx_v], rows_v, sem).wait()
        pltpu.sync_copy(rows_v, out_hbm.at[pl.ds(base, b_per_w)])

    return k(table, idx)


def _tc_math(u_emb, a_ratings, asp):
    B, D = u_emb.shape
    NA = asp.shape[0]
    BB = 2048
    grid = (B // BB,)

    def body(u_ref, a_ref, asp_ref, o_ref):
        w = lax.dot_general(
            u_ref[...], asp_ref[...],
            (((1,), (1,)), ((), ())),
            preferred_element_type=jnp.float32,
        )
        s = jnp.sum(w * a_ref[...], axis=1)
        o_ref[...] = R_MIN + (R_MIN - R_MAX) * ((s - A_MIN) / (A_MAX - A_MIN))

    return pl.pallas_call(
        body,
        grid=grid,
        in_specs=[
            pl.BlockSpec((BB, D), lambda i: (i, 0)),
            pl.BlockSpec((BB, NA), lambda i: (i, 0)),
            pl.BlockSpec((NA, D), lambda i: (0, 0)),
        ],
        out_specs=pl.BlockSpec((BB,), lambda i: (i,)),
        out_shape=jax.ShapeDtypeStruct((B,), jnp.float32),
    )(u_emb, a_ratings, asp)


def kernel(U_ids, A_ratings, users_table, aspects_table):
    idx = U_ids.astype(jnp.int32)
    asp = aspects_table[1:N_ASPECTS]
    u_emb = _sc_gather(users_table, idx)
    return _tc_math(u_emb, A_ratings, asp)


# speedup vs baseline: 1.6932x; 1.6932x over previous
import functools

import jax
import jax.numpy as jnp
from jax import lax
from jax.experimental import pallas as pl
from jax.experimental.pallas import tpu as pltpu
from jax.experimental.pallas import tpu_sc as plsc

N_ASPECTS = 20
EMBED_DIM = 64
BATCH = 16384
A_MIN, A_MAX = 1.0, 5.0
R_MIN, R_MAX = 1.0, 5.0


def _sc_gather(table, idx):
    info = plsc.get_sparse_core_info()
    NC, NS = info.num_cores, info.num_subcores
    NW = NC * NS
    B = idx.shape[0]
    D = table.shape[1]
    b_per_w = B // NW
    mesh = plsc.VectorSubcoreMesh(core_axis_name="c", subcore_axis_name="s")

    @functools.partial(
        pl.kernel,
        mesh=mesh,
        out_type=jax.ShapeDtypeStruct((B, D), jnp.float32),
        scratch_types=[
            pltpu.VMEM((b_per_w,), jnp.int32),
            pltpu.VMEM((b_per_w, D), jnp.float32),
            pltpu.SemaphoreType.DMA,
        ],
    )
    def k(table_hbm, idx_hbm, out_hbm, idx_v, rows_v, sem):
        wid = lax.axis_index("s") * NC + lax.axis_index("c")
        base = wid * b_per_w
        pltpu.sync_copy(idx_hbm.at[pl.ds(base, b_per_w)], idx_v)

        def group(g, carry):
            vec = idx_v[pl.ds(g * 16, 16)]
            for j in range(16):
                s = vec[j]
                pltpu.async_copy(table_hbm.at[s], rows_v.at[g * 16 + j], sem)
            return carry

        lax.fori_loop(0, b_per_w // 16, group, 0)
        pltpu.make_async_copy(
            table_hbm.at[pl.ds(0, b_per_w)], rows_v, sem
        ).wait()
        pltpu.sync_copy(rows_v, out_hbm.at[pl.ds(base, b_per_w)])

    return k(table, idx)


def _tc_math(u_emb, a_ratings, asp):
    B, D = u_emb.shape
    NA = asp.shape[0]
    BB = 2048
    grid = (B // BB,)

    def body(u_ref, a_ref, asp_ref, o_ref):
        w = lax.dot_general(
            u_ref[...], asp_ref[...],
            (((1,), (1,)), ((), ())),
            preferred_element_type=jnp.float32,
        )
        s = jnp.sum(w * a_ref[...], axis=1)
        o_ref[...] = R_MIN + (R_MIN - R_MAX) * ((s - A_MIN) / (A_MAX - A_MIN))

    return pl.pallas_call(
        body,
        grid=grid,
        in_specs=[
            pl.BlockSpec((BB, D), lambda i: (i, 0)),
            pl.BlockSpec((BB, NA), lambda i: (i, 0)),
            pl.BlockSpec((NA, D), lambda i: (0, 0)),
        ],
        out_specs=pl.BlockSpec((BB,), lambda i: (i,)),
        out_shape=jax.ShapeDtypeStruct((B,), jnp.float32),
    )(u_emb, a_ratings, asp)


def kernel(U_ids, A_ratings, users_table, aspects_table):
    idx = U_ids.astype(jnp.int32)
    asp = aspects_table[1:N_ASPECTS]
    u_emb = _sc_gather(users_table, idx)
    return _tc_math(u_emb, A_ratings, asp)


# speedup vs baseline: 2.4069x; 1.4215x over previous
import functools

import jax
import jax.numpy as jnp
from jax import lax
from jax.experimental import pallas as pl
from jax.experimental.pallas import tpu as pltpu
from jax.experimental.pallas import tpu_sc as plsc

N_ASPECTS = 20
EMBED_DIM = 64
BATCH = 16384
A_MIN, A_MAX = 1.0, 5.0
R_MIN, R_MAX = 1.0, 5.0

N_USERS_P1 = 1000001
TAIL_BASE = (N_USERS_P1 - 128) // 128 * 128
SAFE_BASE = TAIL_BASE - 128


def _sc_gather(table_t, idx, tail):
    info = plsc.get_sparse_core_info()
    NC, NS = info.num_cores, info.num_subcores
    NW = NC * NS
    D = table_t.shape[0] * table_t.shape[1]
    B = idx.shape[0]
    b_per_w = B // NW
    n_groups = b_per_w // 4
    mesh = plsc.VectorSubcoreMesh(core_axis_name="c", subcore_axis_name="s")

    @functools.partial(
        pl.kernel,
        mesh=mesh,
        compiler_params=pltpu.CompilerParams(needs_layout_passes=False),
        out_type=jax.ShapeDtypeStruct((D, B), jnp.float32),
        scratch_types=[
            pltpu.VMEM((b_per_w,), jnp.int32),
            pltpu.VMEM((4 * D, 128), jnp.float32),
            pltpu.VMEM((4 * D, 128), jnp.float32),
            pltpu.VMEM((D, b_per_w), jnp.float32),
            pltpu.VMEM((tail.shape[0], D), jnp.float32),
            pltpu.SemaphoreType.DMA,
            pltpu.SemaphoreType.DMA,
        ],
    )
    def k(table_hbm, idx_hbm, tail_hbm, out_hbm,
          idx_v, stage_a, stage_b, rows_v, tail_v, sem_a, sem_b):
        wid = lax.axis_index("s") * NC + lax.axis_index("c")
        base = wid * b_per_w
        pltpu.sync_copy(idx_hbm.at[pl.ds(base, b_per_w)], idx_v)
        pltpu.sync_copy(tail_hbm, tail_v)

        stages = (stage_a, stage_b)
        sems = (sem_a, sem_b)
        lane = jax.lax.broadcasted_iota(jnp.int32, (16,), 0)

        def fire(u, slot, stage, sem):
            blk = jnp.minimum(u // 128, SAFE_BASE // 128) * 128
            for a in range(8):
                pltpu.async_copy(
                    table_hbm.at[a, :, pl.ds(blk, 128)],
                    stage.at[pl.ds(slot * D + a * 8, 8), :],
                    sem,
                )

        def wait_group(stage, sem):
            for slot in range(4):
                for a in range(8):
                    pltpu.make_async_copy(
                        table_hbm.at[0, :, pl.ds(0, 128)],
                        stage.at[pl.ds(slot * D + a * 8, 8), :],
                        sem,
                    ).wait()

        def extract(u, row, slot, stage):
            col = jnp.full((16,), u & 127, jnp.int32)
            rowv = jnp.full((16,), row, jnp.int32)
            for t in range(4):
                vals = plsc.load_gather(stage, [slot * D + t * 16 + lane, col])
                plsc.store_scatter(rows_v, [t * 16 + lane, rowv], vals)

            @pl.when(u >= TAIL_BASE)
            def _():
                trow = jnp.full((16,), u - TAIL_BASE, jnp.int32)
                for t in range(4):
                    tv = plsc.load_gather(tail_v, [trow, t * 16 + lane])
                    plsc.store_scatter(rows_v, [t * 16 + lane, rowv], tv)

        vec0 = idx_v[pl.ds(0, 16)]
        for q in range(2):
            for kk in range(4):
                fire(vec0[4 * q + kk], kk, stages[q], sems[q])

        def body(t, carry):
            vec = idx_v[pl.ds(t * 16, 16)]
            nxt = idx_v[pl.ds(jnp.minimum((t + 1) * 16, b_per_w - 16), 16)]
            for q in range(4):
                g = 4 * t + q
                stage, sem = stages[q % 2], sems[q % 2]
                wait_group(stage, sem)
                for kk in range(4):
                    u = vec[4 * q + kk]
                    extract(u, g * 4 + kk, kk, stage)
                if q < 2:
                    refill = [vec[4 * (q + 2) + kk] for kk in range(4)]
                    for kk in range(4):
                        fire(refill[kk], kk, stage, sem)
                else:
                    @pl.when(t + 1 < n_groups // 4)
                    def _(q=q, stage=stage, sem=sem, nxt=nxt):
                        for kk in range(4):
                            fire(nxt[4 * (q - 2) + kk], kk, stage, sem)
            return carry

        lax.fori_loop(0, n_groups // 4, body, 0)
        pltpu.sync_copy(rows_v, out_hbm.at[:, pl.ds(base, b_per_w)])

    return k(table_t, idx, tail)


def _tc_math(u_emb_t, a_ratings_t, asp):
    D, B = u_emb_t.shape
    NA = asp.shape[0]
    BB = 2048
    grid = (B // BB,)

    def body(u_ref, a_ref, asp_ref, o_ref):
        w = lax.dot_general(
            asp_ref[...], u_ref[...],
            (((1,), (0,)), ((), ())),
            preferred_element_type=jnp.float32,
        )
        s = jnp.sum(w * a_ref[...], axis=0)
        o_ref[...] = R_MIN + (R_MIN - R_MAX) * ((s - A_MIN) / (A_MAX - A_MIN))

    return pl.pallas_call(
        body,
        grid=grid,
        in_specs=[
            pl.BlockSpec((D, BB), lambda i: (0, i)),
            pl.BlockSpec((NA, BB), lambda i: (0, i)),
            pl.BlockSpec((NA, D), lambda i: (0, 0)),
        ],
        out_specs=pl.BlockSpec((BB,), lambda i: (i,)),
        out_shape=jax.ShapeDtypeStruct((B,), jnp.float32),
    )(u_emb_t, a_ratings_t, asp)


def kernel(U_ids, A_ratings, users_table, aspects_table):
    idx = U_ids.astype(jnp.int32)
    table_t = lax.optimization_barrier(users_table.T.reshape(8, 8, N_USERS_P1))
    tail = users_table[TAIL_BASE:]
    a_ratings_t = A_ratings.T
    asp = aspects_table[1:N_ASPECTS]
    u_emb = _sc_gather(table_t, idx, tail)
    return _tc_math(u_emb, a_ratings_t, asp)
